# baseline (speedup 1.0000x reference)
import jax
import jax.numpy as jnp
from jax.experimental import pallas as pl
from jax.experimental.pallas import tpu as pltpu

MEM = 50
LA = 5
EPS = 1e-5

LB = 512
SB = 256
LH = 64
RH = 8
EXTW = SB + LH + RH
RC = 32


def _build_band(mw, lw):
    wm = jnp.sum(mw, axis=1, keepdims=True)
    wf = jnp.sum(lw, axis=1, keepdims=True)
    p = jax.lax.broadcasted_iota(jnp.int32, (SB, EXTW), 0)
    q = jax.lax.broadcasted_iota(jnp.int32, (SB, EXTW), 1)
    d = q - p - (LH - MEM)
    a = jnp.where(d == MEM, 1.0, 0.0)
    for j in range(MEM):
        a = a + jnp.where(d == j, wm[j : j + 1, 0:1], 0.0)
    for k in range(LA):
        a = a + jnp.where(d == MEM + 1 + k, wf[k : k + 1, 0:1], 0.0)
    head = jnp.zeros((SB, EXTW), jnp.float32)
    for j in range(MEM - 1):
        head = head + jnp.where(
            (q == LH + j) & (p > j) & (p < MEM), wm[j : j + 1, 0:1], 0.0
        )
    a0 = jnp.where((p >= MEM) | (q - p >= LH), a, 0.0) + head
    alast = jnp.where(q < LH + SB, a, 0.0)
    return a, a0, alast


def _fused_kernel(
    mw_ref, lw_ref, xl_ref, xc_ref, xr_ref, w_ref, b_ref, g_ref, bt_ref,
    o_ref, a_ref, xb_ref, pre_ref,
):
    b = pl.program_id(0)
    i = pl.program_id(1)
    nlb = pl.num_programs(1)

    @pl.when((b == 0) & (i == 0))
    def _():
        a, a0, alast = _build_band(mw_ref[...], lw_ref[...])
        a_ref[0] = a
        a_ref[1] = a0
        a_ref[2] = alast

    x_ext = jnp.concatenate([xl_ref[0], xc_ref[0], xr_ref[0]], axis=0)

    for k in range(LB // SB):
        m = a_ref[0]
        if k == 0:
            m = jnp.where(i == 0, a_ref[1], m)
        if k == LB // SB - 1:
            m = jnp.where(i == nlb - 1, a_ref[2], m)
        xb_ref[...] = jnp.dot(
            m, x_ext[k * SB : k * SB + EXTW], preferred_element_type=jnp.float32
        )
        rs = jnp.sum(m, axis=1, keepdims=True)
        pre_ref[...] = jax.lax.dot_general(
            xb_ref[...], w_ref[...],
            dimension_numbers=(((1,), (1,)), ((), ())),
            preferred_element_type=jnp.float32,
        )
        for c in range(0, SB, RC):
            z = pre_ref[c : c + RC] + rs[c : c + RC] * b_ref[...]
            mu = jnp.mean(z, axis=1, keepdims=True)
            zc = z - mu
            var = jnp.mean(zc * zc, axis=1, keepdims=True)
            y = zc * jax.lax.rsqrt(var + EPS)
            o_ref[0, k * SB + c : k * SB + c + RC] = y * g_ref[...] + bt_ref[...]


def kernel(x, W_lin, b_lin, mem_w, la_w, gamma, beta):
    B, L, D = x.shape
    H = W_lin.shape[0]
    b2 = b_lin.reshape(1, H)
    g2 = gamma.reshape(1, H)
    bt2 = beta.reshape(1, H)
    nlh = L // LH
    nrh = L // RH

    return pl.pallas_call(
        _fused_kernel,
        grid=(B, L // LB),
        in_specs=[
            pl.BlockSpec((MEM, H), lambda b, i: (0, 0)),
            pl.BlockSpec((LA, H), lambda b, i: (0, 0)),
            pl.BlockSpec(
                (1, LH, D),
                lambda b, i: (b, jnp.maximum(i * (LB // LH) - 1, 0), 0),
            ),
            pl.BlockSpec((1, LB, D), lambda b, i: (b, i, 0)),
            pl.BlockSpec(
                (1, RH, D),
                lambda b, i: (b, jnp.minimum((i + 1) * (LB // RH), nrh - 1), 0),
            ),
            pl.BlockSpec((H, D), lambda b, i: (0, 0)),
            pl.BlockSpec((1, H), lambda b, i: (0, 0)),
            pl.BlockSpec((1, H), lambda b, i: (0, 0)),
            pl.BlockSpec((1, H), lambda b, i: (0, 0)),
        ],
        out_specs=pl.BlockSpec((1, LB, H), lambda b, i: (b, i, 0)),
        out_shape=jax.ShapeDtypeStruct((B, L, H), jnp.float32),
        scratch_shapes=[
            pltpu.VMEM((3, SB, EXTW), jnp.float32),
            pltpu.VMEM((SB, D), jnp.float32),
            pltpu.VMEM((SB, H), jnp.float32),
        ],
        compiler_params=pltpu.CompilerParams(
            dimension_semantics=("arbitrary", "arbitrary"),
            vmem_limit_bytes=44 * 1024 * 1024,
        ),
        name="dfsmn_fused",
    )(mem_w, la_w, x, x, x, W_lin, b2, g2, bt2)


# speedup vs baseline: 1.0045x; 1.0045x over previous
import jax
import jax.numpy as jnp
from jax.experimental import pallas as pl
from jax.experimental.pallas import tpu as pltpu

MEM = 50
LA = 5
EPS = 1e-5

LB = 1024
SB = 256
LH = 64
RH = 8
EXTW = SB + LH + RH
RC = 32


def _build_band(mw, lw):
    wm = jnp.sum(mw, axis=1, keepdims=True)
    wf = jnp.sum(lw, axis=1, keepdims=True)
    p = jax.lax.broadcasted_iota(jnp.int32, (SB, EXTW), 0)
    q = jax.lax.broadcasted_iota(jnp.int32, (SB, EXTW), 1)
    d = q - p - (LH - MEM)
    a = jnp.where(d == MEM, 1.0, 0.0)
    for j in range(MEM):
        a = a + jnp.where(d == j, wm[j : j + 1, 0:1], 0.0)
    for k in range(LA):
        a = a + jnp.where(d == MEM + 1 + k, wf[k : k + 1, 0:1], 0.0)
    head = jnp.zeros((SB, EXTW), jnp.float32)
    for j in range(MEM - 1):
        head = head + jnp.where(
            (q == LH + j) & (p > j) & (p < MEM), wm[j : j + 1, 0:1], 0.0
        )
    a0 = jnp.where((p >= MEM) | (q - p >= LH), a, 0.0) + head
    alast = jnp.where(q < LH + SB, a, 0.0)
    return a, a0, alast


def _fused_kernel(
    mw_ref, lw_ref, xl_ref, xc_ref, xr_ref, w_ref, b_ref, g_ref, bt_ref,
    o_ref, a_ref, xb_ref, pre_ref,
):
    b = pl.program_id(0)
    i = pl.program_id(1)
    nlb = pl.num_programs(1)
    nsub = LB // SB

    @pl.when((b == 0) & (i == 0))
    def _():
        a, a0, alast = _build_band(mw_ref[...], lw_ref[...])
        a_ref[0] = a.astype(jnp.bfloat16)
        a_ref[1] = a0.astype(jnp.bfloat16)
        a_ref[2] = alast.astype(jnp.bfloat16)

    x_ext = jnp.concatenate(
        [xl_ref[0], xc_ref[0], xr_ref[0]], axis=0
    ).astype(jnp.bfloat16)

    for k in range(nsub):
        if k == 0:
            idx = jnp.where(i == 0, 1, 0)
        elif k == nsub - 1:
            idx = jnp.where(i == nlb - 1, 2, 0)
        else:
            idx = 0
        m = a_ref[idx]
        xb_ref[k % 2] = jnp.dot(
            m, x_ext[k * SB : k * SB + EXTW], preferred_element_type=jnp.float32
        ).astype(jnp.bfloat16)
        rs = jnp.sum(m.astype(jnp.float32), axis=1, keepdims=True)
        pre_ref[k % 2] = jax.lax.dot_general(
            xb_ref[k % 2], w_ref[...],
            dimension_numbers=(((1,), (1,)), ((), ())),
            preferred_element_type=jnp.float32,
        )
        for c in range(0, SB, RC):
            z = pre_ref[k % 2, c : c + RC] + rs[c : c + RC] * b_ref[...]
            mu = jnp.mean(z, axis=1, keepdims=True)
            zc = z - mu
            var = jnp.mean(zc * zc, axis=1, keepdims=True)
            y = zc * jax.lax.rsqrt(var + EPS)
            o_ref[0, k * SB + c : k * SB + c + RC] = y * g_ref[...] + bt_ref[...]


def kernel(x, W_lin, b_lin, mem_w, la_w, gamma, beta):
    B, L, D = x.shape
    H = W_lin.shape[0]
    w_bf = W_lin.astype(jnp.bfloat16)
    b2 = b_lin.reshape(1, H)
    g2 = gamma.reshape(1, H)
    bt2 = beta.reshape(1, H)
    nrh = L // RH

    return pl.pallas_call(
        _fused_kernel,
        grid=(B, L // LB),
        in_specs=[
            pl.BlockSpec((MEM, H), lambda b, i: (0, 0)),
            pl.BlockSpec((LA, H), lambda b, i: (0, 0)),
            pl.BlockSpec(
                (1, LH, D),
                lambda b, i: (b, jnp.maximum(i * (LB // LH) - 1, 0), 0),
            ),
            pl.BlockSpec((1, LB, D), lambda b, i: (b, i, 0)),
            pl.BlockSpec(
                (1, RH, D),
                lambda b, i: (b, jnp.minimum((i + 1) * (LB // RH), nrh - 1), 0),
            ),
            pl.BlockSpec((H, D), lambda b, i: (0, 0)),
            pl.BlockSpec((1, H), lambda b, i: (0, 0)),
            pl.BlockSpec((1, H), lambda b, i: (0, 0)),
            pl.BlockSpec((1, H), lambda b, i: (0, 0)),
        ],
        out_specs=pl.BlockSpec((1, LB, H), lambda b, i: (b, i, 0)),
        out_shape=jax.ShapeDtypeStruct((B, L, H), jnp.float32),
        scratch_shapes=[
            pltpu.VMEM((3, SB, EXTW), jnp.bfloat16),
            pltpu.VMEM((2, SB, D), jnp.bfloat16),
            pltpu.VMEM((2, SB, H), jnp.float32),
        ],
        compiler_params=pltpu.CompilerParams(
            dimension_semantics=("arbitrary", "arbitrary"),
            vmem_limit_bytes=48 * 1024 * 1024,
        ),
        name="dfsmn_fused",
    )(mem_w, la_w, x, x, x, w_bf, b2, g2, bt2)
